# baseline (speedup 1.0000x reference)
import functools

import jax
import jax.numpy as jnp
from jax import lax
from jax.experimental import pallas as pl
from jax.experimental.pallas import tpu as pltpu
from jax.experimental.pallas import tpu_sc as plsc

EMBED_DIM = 64
NUM_CORES = 2
NUM_SUBCORES = 16
NUM_WORKERS = NUM_CORES * NUM_SUBCORES
CH = 256


@jax.jit
def _gather_t(ids_t, table2):
    T, S = ids_t.shape
    s_per_w = S // NUM_WORKERS
    halves = s_per_w // CH
    n_chunks = T * halves
    mesh = plsc.VectorSubcoreMesh(core_axis_name="c", subcore_axis_name="s")

    @functools.partial(
        pl.kernel,
        mesh=mesh,
        out_type=jax.ShapeDtypeStruct((T, EMBED_DIM, S), jnp.float32),
        scratch_types=[
            pltpu.VMEM((CH,), jnp.int32),
            pltpu.VMEM((CH,), jnp.int32),
            pltpu.VMEM((CH,), jnp.int32),
            pltpu.VMEM((CH,), jnp.int32),
            pltpu.VMEM((CH,), jnp.int32),
            pltpu.VMEM((CH,), jnp.int32),
            pltpu.VMEM((CH, 2 * EMBED_DIM), jnp.float32),
            pltpu.VMEM((CH, 2 * EMBED_DIM), jnp.float32),
            pltpu.VMEM((EMBED_DIM, CH), jnp.float32),
            pltpu.VMEM((EMBED_DIM, CH), jnp.float32),
            pltpu.SemaphoreType.DMA,
            pltpu.SemaphoreType.DMA,
            pltpu.SemaphoreType.DMA,
            pltpu.SemaphoreType.DMA,
        ],
        compiler_params=pltpu.CompilerParams(needs_layout_passes=False),
    )
    def k(ids_hbm, table_hbm, out_hbm,
          idx0, idx1, pix0, pix1, par0, par1, rows0, rows1, tr0, tr1,
          gsem0, gsem1, wsem0, wsem1):
        wid = lax.axis_index("s") * NUM_CORES + lax.axis_index("c")
        s0 = pl.multiple_of(wid * s_per_w, 8)
        idx = (idx0, idx1)
        pix = (pix0, pix1)
        par = (par0, par1)
        rows = (rows0, rows1)
        tr = (tr0, tr1)
        gsem = (gsem0, gsem1)
        wsem = (wsem0, wsem1)
        iota = lax.iota(jnp.int32, 16)
        rots = [jnp.where(iota + r < 16, iota + r, iota + r - 16)
                for r in range(16)]

        def ids_slice(c):
            t = c // halves
            return ids_hbm.at[t, pl.ds(s0 + (c % halves) * CH, CH)]

        def out_slice(c):
            t = c // halves
            return out_hbm.at[t, :, pl.ds(s0 + (c % halves) * CH, CH)]

        def prep_indices(b):
            for k16 in range(CH // 16):
                v = idx[b][pl.ds(16 * k16, 16)]
                pix[b][pl.ds(16 * k16, 16)] = lax.shift_right_logical(v, 1)
                par[b][pl.ds(16 * k16, 16)] = lax.shift_left(
                    lax.bitwise_and(v, 1), 6)

        def transpose(b):
            rbuf, tbuf, pbuf = rows[b], tr[b], par[b]

            @plsc.parallel_loop(0, CH, 16, unroll=4)
            def _(sb):
                srow = sb + iota
                pv = pbuf[pl.ds(sb, 16)]
                for d0 in range(0, EMBED_DIM, 16):
                    for r in range(0, 16):
                        dcol = d0 + rots[r]
                        vec = plsc.load_gather(rbuf, [srow, pv + dcol])
                        plsc.store_scatter(tbuf, [dcol, srow], vec)

        def stage(c, b):
            pltpu.make_async_copy(
                table_hbm.at[pix[b]], rows[b], gsem[b]
            ).wait()

            @pl.when(c + 1 < n_chunks)
            def _():
                o = 1 - b
                pltpu.sync_copy(ids_slice(c + 1), idx[o])
                prep_indices(o)
                pltpu.async_copy(table_hbm.at[pix[o]], rows[o], gsem[o])

            @pl.when(c >= 2)
            def _():
                pltpu.make_async_copy(tr[b], out_slice(c), wsem[b]).wait()

            transpose(b)
            pltpu.async_copy(tr[b], out_slice(c), wsem[b])

        pltpu.sync_copy(ids_slice(0), idx[0])
        prep_indices(0)
        pltpu.async_copy(table_hbm.at[pix[0]], rows[0], gsem[0])

        def body(i, carry):
            stage(2 * i, 0)
            stage(2 * i + 1, 1)
            return carry

        lax.fori_loop(0, n_chunks // 2, body, 0)

        pltpu.make_async_copy(tr[0], out_slice(n_chunks - 2), wsem[0]).wait()
        pltpu.make_async_copy(tr[1], out_slice(n_chunks - 1), wsem[1]).wait()

    return k(ids_t, table2)


def kernel(token_ids, embedding):
    table2 = embedding.reshape(500000, 2 * EMBED_DIM)
    out_t = _gather_t(token_ids.T, table2)
    return out_t.transpose(2, 0, 1)


# speedup vs baseline: 1.0471x; 1.0471x over previous
import functools

import jax
import jax.numpy as jnp
from jax import lax
from jax.experimental import pallas as pl
from jax.experimental.pallas import tpu as pltpu
from jax.experimental.pallas import tpu_sc as plsc

EMBED_DIM = 64
NUM_CORES = 2
NUM_SUBCORES = 16
NUM_WORKERS = NUM_CORES * NUM_SUBCORES
CH = 256


@jax.jit
def _gather_t(ids_t, table2):
    T, S = ids_t.shape
    s_per_w = S // NUM_WORKERS
    halves = s_per_w // CH
    n_chunks = T * halves
    mesh = plsc.VectorSubcoreMesh(core_axis_name="c", subcore_axis_name="s")

    @functools.partial(
        pl.kernel,
        mesh=mesh,
        out_type=jax.ShapeDtypeStruct((T, EMBED_DIM, S), jnp.float32),
        scratch_types=[
            pltpu.VMEM((CH,), jnp.int32),
            pltpu.VMEM((CH,), jnp.int32),
            pltpu.VMEM((CH,), jnp.int32),
            pltpu.VMEM((CH,), jnp.int32),
            pltpu.VMEM((CH,), jnp.int32),
            pltpu.VMEM((CH,), jnp.int32),
            pltpu.VMEM((CH, 2 * EMBED_DIM), jnp.float32),
            pltpu.VMEM((CH, 2 * EMBED_DIM), jnp.float32),
            pltpu.VMEM((EMBED_DIM, CH), jnp.float32),
            pltpu.VMEM((EMBED_DIM, CH), jnp.float32),
            pltpu.SemaphoreType.DMA,
            pltpu.SemaphoreType.DMA,
            pltpu.SemaphoreType.DMA,
            pltpu.SemaphoreType.DMA,
        ],
        compiler_params=pltpu.CompilerParams(needs_layout_passes=False),
    )
    def k(ids_hbm, table_hbm, out_hbm,
          idx0, idx1, pix0, pix1, par0, par1, rows0, rows1, tr0, tr1,
          gsem0, gsem1, wsem0, wsem1):
        wid = lax.axis_index("s") * NUM_CORES + lax.axis_index("c")
        s0 = pl.multiple_of(wid * s_per_w, 8)
        idx = (idx0, idx1)
        pix = (pix0, pix1)
        par = (par0, par1)
        rows = (rows0, rows1)
        tr = (tr0, tr1)
        gsem = (gsem0, gsem1)
        wsem = (wsem0, wsem1)
        iota = lax.iota(jnp.int32, 16)
        rots = [jnp.where(iota + r < 16, iota + r, iota + r - 16)
                for r in range(16)]

        def ids_slice(c):
            t = c // halves
            return ids_hbm.at[t, pl.ds(s0 + (c % halves) * CH, CH)]

        def out_slice(c):
            t = c // halves
            return out_hbm.at[t, :, pl.ds(s0 + (c % halves) * CH, CH)]

        def prep_indices(b):
            for k16 in range(CH // 16):
                v = idx[b][pl.ds(16 * k16, 16)]
                pix[b][pl.ds(16 * k16, 16)] = lax.shift_right_logical(v, 1)
                par[b][pl.ds(16 * k16, 16)] = lax.shift_left(
                    lax.bitwise_and(v, 1), 6)

        def transpose(b):
            rbuf, tbuf, pbuf = rows[b], tr[b], par[b]

            @plsc.parallel_loop(0, CH, 16, unroll=2)
            def _(sb):
                srow = sb + iota
                pv = pbuf[pl.ds(sb, 16)]
                for d0 in range(0, EMBED_DIM, 16):
                    for r in range(0, 16):
                        dcol = d0 + rots[r]
                        vec = plsc.load_gather(rbuf, [srow, pv + dcol])
                        plsc.store_scatter(tbuf, [dcol, srow], vec)

        def stage(c, b):
            pltpu.make_async_copy(
                table_hbm.at[pix[b]], rows[b], gsem[b]
            ).wait()

            @pl.when(c + 1 < n_chunks)
            def _():
                o = 1 - b
                pltpu.sync_copy(ids_slice(c + 1), idx[o])
                prep_indices(o)
                pltpu.async_copy(table_hbm.at[pix[o]], rows[o], gsem[o])

            @pl.when(c >= 2)
            def _():
                pltpu.make_async_copy(tr[b], out_slice(c), wsem[b]).wait()

            transpose(b)
            pltpu.async_copy(tr[b], out_slice(c), wsem[b])

        pltpu.sync_copy(ids_slice(0), idx[0])
        prep_indices(0)
        pltpu.async_copy(table_hbm.at[pix[0]], rows[0], gsem[0])

        def body(i, carry):
            stage(2 * i, 0)
            stage(2 * i + 1, 1)
            return carry

        lax.fori_loop(0, n_chunks // 2, body, 0)

        pltpu.make_async_copy(tr[0], out_slice(n_chunks - 2), wsem[0]).wait()
        pltpu.make_async_copy(tr[1], out_slice(n_chunks - 1), wsem[1]).wait()

    return k(ids_t, table2)


def kernel(token_ids, embedding):
    table2 = embedding.reshape(500000, 2 * EMBED_DIM)
    out_t = _gather_t(token_ids.T, table2)
    return out_t.transpose(2, 0, 1)
